# Initial kernel scaffold; baseline (speedup 1.0000x reference)
import jax
import jax.numpy as jnp
from jax.experimental import pallas as pl


def kernel(ent_table, W, b, triple_ids):
    raise NotImplementedError("write your pallas kernel here")


# speedup vs baseline: 4.4884x; 4.4884x over previous
import functools

import jax
import jax.numpy as jnp
from jax import lax
from jax.experimental import pallas as pl
from jax.experimental.pallas import tpu as pltpu
from jax.experimental.pallas import tpu_sc as plsc

PRETRAIN_DIM = 128
DIM_LLM = 768
NUM_PREFIX = 1


def _make_sc_gather(B, D):
    info = plsc.get_sparse_core_info()
    NC, NS = info.num_cores, info.num_subcores
    NW = NC * NS
    b_per_w = B // NW
    CH = 128
    n_ch = b_per_w // CH
    mesh = plsc.VectorSubcoreMesh(core_axis_name="c", subcore_axis_name="s")

    @functools.partial(
        pl.kernel,
        mesh=mesh,
        out_type=jax.ShapeDtypeStruct((B, D), jnp.float32),
        scratch_types=[
            pltpu.VMEM((n_ch, CH), jnp.int32),
            pltpu.VMEM((2, CH, D), jnp.float32),
            pltpu.SemaphoreType.DMA,
            pltpu.SemaphoreType.DMA,
        ],
    )
    def gather_kernel(table_hbm, idx_hbm, out_hbm, idx_v, rows_v, gsem, osem):
        wid = lax.axis_index("s") * NC + lax.axis_index("c")
        base = wid * b_per_w
        pltpu.sync_copy(idx_hbm.at[wid], idx_v)
        for j in range(n_ch):
            buf = j % 2
            pltpu.async_copy(table_hbm.at[idx_v.at[j]], rows_v.at[buf], gsem).wait()
            pltpu.async_copy(
                rows_v.at[buf], out_hbm.at[pl.ds(base + j * CH, CH)], osem
            ).wait()

    return gather_kernel, NW, n_ch, CH


def _adapter_matmul(emb, wt, b):
    M, K = emb.shape
    N = wt.shape[1]
    BM = 2048

    def mm_kernel(emb_ref, wt_ref, b_ref, out_ref):
        out_ref[...] = (
            jnp.dot(emb_ref[...], wt_ref[...], preferred_element_type=jnp.float32)
            + b_ref[...]
        )

    return pl.pallas_call(
        mm_kernel,
        grid=(M // BM,),
        in_specs=[
            pl.BlockSpec((BM, K), lambda i: (i, 0)),
            pl.BlockSpec((K, N), lambda i: (0, 0)),
            pl.BlockSpec((1, N), lambda i: (0, 0)),
        ],
        out_specs=pl.BlockSpec((BM, N), lambda i: (i, 0)),
        out_shape=jax.ShapeDtypeStruct((M, N), jnp.float32),
    )(emb, wt, b.reshape(1, N))


def kernel(ent_table, W, b, triple_ids):
    ids = triple_ids.reshape(-1).astype(jnp.int32)
    B = ids.shape[0]
    D = ent_table.shape[1]
    gather_fn, NW, n_ch, CH = _make_sc_gather(B, D)
    idx = ids.reshape(NW, n_ch, CH)
    emb = gather_fn(ent_table, idx)
    proj = _adapter_matmul(emb, W.T, b)
    return proj.reshape(-1, NUM_PREFIX, DIM_LLM)


# speedup vs baseline: 7.7268x; 1.7215x over previous
import functools

import jax
import jax.numpy as jnp
from jax import lax
from jax.experimental import pallas as pl
from jax.experimental.pallas import tpu as pltpu
from jax.experimental.pallas import tpu_sc as plsc

PRETRAIN_DIM = 128
DIM_LLM = 768
NUM_PREFIX = 1


def _make_sc_gather(B, D):
    info = plsc.get_sparse_core_info()
    NC, NS = info.num_cores, info.num_subcores
    NW = NC * NS
    b_per_w = B // NW
    CH = 128
    n_ch = b_per_w // CH
    mesh = plsc.VectorSubcoreMesh(core_axis_name="c", subcore_axis_name="s")

    @functools.partial(
        pl.kernel,
        mesh=mesh,
        out_type=jax.ShapeDtypeStruct((B, D), jnp.float32),
        scratch_types=[
            pltpu.VMEM((n_ch, CH), jnp.int32),
            pltpu.VMEM((2, CH, D), jnp.float32),
            pltpu.SemaphoreType.DMA,
            pltpu.SemaphoreType.DMA,
        ],
    )
    def gather_kernel(table_hbm, idx_hbm, out_hbm, idx_v, rows_v, gsem, osem):
        wid = lax.axis_index("s") * NC + lax.axis_index("c")
        base = wid * b_per_w
        pltpu.sync_copy(idx_hbm.at[wid], idx_v)
        for j in range(n_ch):
            buf = j % 2
            pltpu.async_copy(table_hbm.at[idx_v.at[j]], rows_v.at[buf], gsem).wait()
            pltpu.async_copy(
                rows_v.at[buf], out_hbm.at[pl.ds(base + j * CH, CH)], osem
            ).wait()

    return gather_kernel, NW, n_ch, CH


def _adapter_matmul(emb, wt, b):
    M, K = emb.shape
    N = wt.shape[1]
    BM = 2048

    def mm_kernel(emb_ref, wt_ref, b_ref, out_ref):
        out_ref[...] = (
            jnp.dot(emb_ref[...], wt_ref[...], preferred_element_type=jnp.float32)
            + b_ref[...]
        )[:, None, :]

    return pl.pallas_call(
        mm_kernel,
        grid=(M // BM,),
        in_specs=[
            pl.BlockSpec((BM, K), lambda i: (i, 0)),
            pl.BlockSpec((K, N), lambda i: (0, 0)),
            pl.BlockSpec((1, N), lambda i: (0, 0)),
        ],
        out_specs=pl.BlockSpec((BM, 1, N), lambda i: (i, 0, 0)),
        out_shape=jax.ShapeDtypeStruct((M, 1, N), jnp.float32),
    )(emb, wt, b.reshape(1, N))


def kernel(ent_table, W, b, triple_ids):
    ids = triple_ids.reshape(-1).astype(jnp.int32)
    B = ids.shape[0]
    D = ent_table.shape[1]
    gather_fn, NW, n_ch, CH = _make_sc_gather(B, D)
    idx = ids.reshape(NW, n_ch, CH)
    emb = gather_fn(ent_table, idx)
    return _adapter_matmul(emb, W.T, b)


# speedup vs baseline: 7.9051x; 1.0231x over previous
import functools

import jax
import jax.numpy as jnp
from jax import lax
from jax.experimental import pallas as pl
from jax.experimental.pallas import tpu as pltpu
from jax.experimental.pallas import tpu_sc as plsc

PRETRAIN_DIM = 128
DIM_LLM = 768
NUM_PREFIX = 1
P_CHUNKS = 2
BM = 2048


def _make_sc_gather(B, D):
    info = plsc.get_sparse_core_info()
    NC, NS = info.num_cores, info.num_subcores
    NW = NC * NS
    b_per_w = B // NW
    CH = 128
    n_ch = b_per_w // CH
    mesh = plsc.VectorSubcoreMesh(core_axis_name="c", subcore_axis_name="s")

    @functools.partial(
        pl.kernel,
        mesh=mesh,
        out_type=jax.ShapeDtypeStruct((B, D), jnp.float32),
        scratch_types=[
            pltpu.VMEM((n_ch, CH), jnp.int32),
            pltpu.VMEM((2, CH, D), jnp.float32),
            pltpu.SemaphoreType.DMA,
            pltpu.SemaphoreType.DMA,
        ],
    )
    def gather_kernel(table_hbm, idx_hbm, out_hbm, idx_v, rows_v, gsem, osem):
        wid = lax.axis_index("s") * NC + lax.axis_index("c")
        base = wid * b_per_w
        pltpu.sync_copy(idx_hbm.at[wid], idx_v)
        for j in range(n_ch):
            buf = j % 2
            pltpu.async_copy(table_hbm.at[idx_v.at[j]], rows_v.at[buf], gsem).wait()
            pltpu.async_copy(
                rows_v.at[buf], out_hbm.at[pl.ds(base + j * CH, CH)], osem
            ).wait()

    return gather_kernel, NW, n_ch, CH


def _matmul_chunk(emb, wt, b2, prev_out, p, M):
    Bc, K = emb.shape
    N = wt.shape[1]
    blk_off = p * (Bc // BM)

    def mm_kernel(emb_ref, wt_ref, b_ref, prev_ref, out_ref):
        out_ref[...] = (
            jnp.dot(emb_ref[...], wt_ref[...], preferred_element_type=jnp.float32)
            + b_ref[...]
        )[:, None, :]

    def mm_kernel_first(emb_ref, wt_ref, b_ref, out_ref):
        out_ref[...] = (
            jnp.dot(emb_ref[...], wt_ref[...], preferred_element_type=jnp.float32)
            + b_ref[...]
        )[:, None, :]

    in_specs = [
        pl.BlockSpec((BM, K), lambda i: (i, 0)),
        pl.BlockSpec((K, N), lambda i: (0, 0)),
        pl.BlockSpec((1, N), lambda i: (0, 0)),
    ]
    out_spec = pl.BlockSpec((BM, 1, N), lambda i, _o=blk_off: (i + _o, 0, 0))
    out_shape = jax.ShapeDtypeStruct((M, 1, N), jnp.float32)
    if prev_out is None:
        return pl.pallas_call(
            mm_kernel_first,
            grid=(Bc // BM,),
            in_specs=in_specs,
            out_specs=out_spec,
            out_shape=out_shape,
        )(emb, wt, b2)
    return pl.pallas_call(
        mm_kernel,
        grid=(Bc // BM,),
        in_specs=in_specs + [pl.BlockSpec(memory_space=pltpu.HBM)],
        out_specs=out_spec,
        out_shape=out_shape,
        input_output_aliases={3: 0},
    )(emb, wt, b2, prev_out)


def kernel(ent_table, W, b, triple_ids):
    ids = triple_ids.reshape(-1).astype(jnp.int32)
    B = ids.shape[0]
    D = ent_table.shape[1]
    Bc = B // P_CHUNKS
    gather_fn, NW, n_ch, CH = _make_sc_gather(Bc, D)
    wt = W.T
    b2 = b.reshape(1, -1)
    out = None
    for p in range(P_CHUNKS):
        idx = lax.slice(ids, (p * Bc,), ((p + 1) * Bc,)).reshape(NW, n_ch, CH)
        emb = gather_fn(ent_table, idx)
        out = _matmul_chunk(emb, wt, b2, out, p, B)
    return out
